# baseline (speedup 1.0000x reference)
import jax
import jax.numpy as jnp
from jax.experimental import pallas as pl
from jax.experimental.pallas import tpu as pltpu


def _body(x_ref, w_ref, b_ref, s_ref, st_ref, o_ref, *, d, n, k):
    m = w_ref.shape[0]
    X = x_ref[0]
    Y = jnp.dot(w_ref[...], X, preferred_element_type=jnp.float32) + b_ref[...]

    sub = jax.lax.broadcasted_iota(jnp.int32, (m, n), 0)
    s_row = s_ref[0]
    s_col = st_ref[0]
    s_b = jnp.broadcast_to(s_row, (m, n))
    G = jnp.where(sub < d, jnp.maximum(Y, 0.0), jnp.where(sub == d, s_b, 0.0))

    isub = jax.lax.broadcasted_iota(jnp.int32, (n, n), 0)
    jlan = jax.lax.broadcasted_iota(jnp.int32, (n, n), 1)
    sr = jnp.broadcast_to(s_row, (n, n))
    sc = jnp.broadcast_to(s_col, (n, n))
    cmp = (sr > sc) | ((sr == sc) & (jlan < isub))
    rank = jnp.sum(cmp.astype(jnp.float32), axis=1, keepdims=True)

    kf = jax.lax.broadcasted_iota(jnp.int32, (n, n), 1).astype(jnp.float32)
    PT = (jnp.broadcast_to(rank, (n, n)) == kf).astype(jnp.float32)
    O = jnp.dot(G, PT, preferred_element_type=jnp.float32)

    kmask = jax.lax.broadcasted_iota(jnp.int32, (m, n), 1) < k
    Om = jnp.where(kmask, O, 0.0)
    sq = jnp.sum(Om * Om, axis=1, keepdims=True)
    den = jnp.maximum(jnp.sqrt(sq), 1e-12)
    o_ref[0] = jnp.where(sub < d, O / den, O)


def kernel(features, att_w, att_b, proj_w, proj_b, bn_gamma, bn_beta,
           bn_mean, bn_var, num_keypoints):
    B, C, H, W = features.shape
    D = proj_w.shape[0]
    N = H * W
    K = min(1000, N)
    eps = 1e-5
    M = D + 8

    x = features.reshape(B, C, N)
    att = jax.nn.sigmoid(jnp.einsum('bchw,oc->bohw', features, att_w)
                         + att_b[None, :, None, None])
    s3 = att.reshape(B, 1, N)
    st3 = att.reshape(B, N, 1)
    scale = bn_gamma / jnp.sqrt(bn_var + eps)
    w_loc = proj_w * scale[:, None]
    b_loc = (proj_b - bn_mean) * scale + bn_beta
    w_all = jnp.concatenate(
        [w_loc, att_w, jnp.zeros((M - D - 1, C), jnp.float32)], axis=0)
    b_all = jnp.concatenate(
        [b_loc, att_b, jnp.zeros((M - D - 1,), jnp.float32)], axis=0)
    b_all = jnp.broadcast_to(b_all[:, None], (M, N))

    import functools
    body = functools.partial(_body, d=D, n=N, k=K)
    out = pl.pallas_call(
        body,
        grid=(B,),
        in_specs=[
            pl.BlockSpec((1, C, N), lambda b: (b, 0, 0)),
            pl.BlockSpec((M, C), lambda b: (0, 0)),
            pl.BlockSpec((M, N), lambda b: (0, 0)),
            pl.BlockSpec((1, 1, N), lambda b: (b, 0, 0)),
            pl.BlockSpec((1, N, 1), lambda b: (b, 0, 0)),
        ],
        out_specs=pl.BlockSpec((1, M, N), lambda b: (b, 0, 0)),
        out_shape=jax.ShapeDtypeStruct((B, M, N), jnp.float32),
        compiler_params=pltpu.CompilerParams(
            dimension_semantics=("arbitrary",)),
    )(x, w_all, b_all, s3, st3)

    local_desc = out[:, :D, :K]
    scores = out[:, D, :K]
    return (local_desc, scores)


# speedup vs baseline: 1.0059x; 1.0059x over previous
import jax
import jax.numpy as jnp
from jax.experimental import pallas as pl
from jax.experimental.pallas import tpu as pltpu


def _body(x_ref, w_ref, b_ref, s_ref, st_ref, od_ref, os_ref, *, d, n, k):
    m = w_ref.shape[0]
    X = x_ref[0]
    Y = jnp.dot(w_ref[...], X, preferred_element_type=jnp.float32) + b_ref[...]

    sub = jax.lax.broadcasted_iota(jnp.int32, (m, n), 0)
    s_row = s_ref[0]
    s_col = st_ref[0]
    s_b = jnp.broadcast_to(s_row, (m, n))
    G = jnp.where(sub < d, jnp.maximum(Y, 0.0), jnp.where(sub == d, s_b, 0.0))

    isub = jax.lax.broadcasted_iota(jnp.int32, (n, n), 0)
    jlan = jax.lax.broadcasted_iota(jnp.int32, (n, n), 1)
    sr = jnp.broadcast_to(s_row, (n, n))
    sc = jnp.broadcast_to(s_col, (n, n))
    cmp = (sr > sc) | ((sr == sc) & (jlan < isub))
    rank = jnp.sum(cmp.astype(jnp.float32), axis=1, keepdims=True)

    kf = jax.lax.broadcasted_iota(jnp.int32, (n, n), 1).astype(jnp.float32)
    PT = (jnp.broadcast_to(rank, (n, n)) == kf).astype(jnp.float32)
    O = jnp.dot(G, PT, preferred_element_type=jnp.float32)

    kmask = jax.lax.broadcasted_iota(jnp.int32, (m, n), 1) < k
    Om = jnp.where(kmask, O, 0.0)
    sq = jnp.sum(Om * Om, axis=1, keepdims=True)
    den = jnp.maximum(jnp.sqrt(sq), 1e-12)
    desc = O[0:d, :] / den[0:d, :]
    od_ref[0] = desc[:, 0:k]
    os_ref[0] = O[d:d + 1, 0:k]


def kernel(features, att_w, att_b, proj_w, proj_b, bn_gamma, bn_beta,
           bn_mean, bn_var, num_keypoints):
    B, C, H, W = features.shape
    D = proj_w.shape[0]
    N = H * W
    K = min(1000, N)
    eps = 1e-5
    M = D + 8

    x = features.reshape(B, C, N)
    att = jax.nn.sigmoid(jnp.einsum('bchw,oc->bohw', features, att_w)
                         + att_b[None, :, None, None])
    s3 = att.reshape(B, 1, N)
    st3 = att.reshape(B, N, 1)
    scale = bn_gamma / jnp.sqrt(bn_var + eps)
    w_loc = proj_w * scale[:, None]
    b_loc = (proj_b - bn_mean) * scale + bn_beta
    w_all = jnp.concatenate(
        [w_loc, att_w, jnp.zeros((M - D - 1, C), jnp.float32)], axis=0)
    b_all = jnp.concatenate(
        [b_loc, att_b, jnp.zeros((M - D - 1,), jnp.float32)], axis=0)
    b_all = jnp.broadcast_to(b_all[:, None], (M, N))

    import functools
    body = functools.partial(_body, d=D, n=N, k=K)
    out = pl.pallas_call(
        body,
        grid=(B,),
        in_specs=[
            pl.BlockSpec((1, C, N), lambda b: (b, 0, 0)),
            pl.BlockSpec((M, C), lambda b: (0, 0)),
            pl.BlockSpec((M, N), lambda b: (0, 0)),
            pl.BlockSpec((1, 1, N), lambda b: (b, 0, 0)),
            pl.BlockSpec((1, N, 1), lambda b: (b, 0, 0)),
        ],
        out_specs=[
            pl.BlockSpec((1, D, K), lambda b: (b, 0, 0)),
            pl.BlockSpec((1, 1, K), lambda b: (b, 0, 0)),
        ],
        out_shape=[
            jax.ShapeDtypeStruct((B, D, K), jnp.float32),
            jax.ShapeDtypeStruct((B, 1, K), jnp.float32),
        ],
        compiler_params=pltpu.CompilerParams(
            dimension_semantics=("arbitrary",)),
    )(x, w_all, b_all, s3, st3)

    local_desc, scores3 = out
    return (local_desc, scores3[:, 0, :])


# speedup vs baseline: 1.1175x; 1.1110x over previous
import functools

import jax
import jax.numpy as jnp
from jax.experimental import pallas as pl
from jax.experimental.pallas import tpu as pltpu


def _body(x_ref, w_ref, b_ref, s_ref, od_ref, os_ref, *, d, n, k):
    m = w_ref.shape[0]
    X = x_ref[0]
    Y = jnp.dot(w_ref[...], X, preferred_element_type=jnp.float32) + b_ref[...]

    sub = jax.lax.broadcasted_iota(jnp.int32, (m, n), 0)
    s_row = s_ref[0]
    s_col = jnp.transpose(jnp.broadcast_to(s_row, (8, n)), (1, 0))[:, 0:1]
    s_b = jnp.broadcast_to(s_row, (m, n))
    G = jnp.where(sub < d, jnp.maximum(Y, 0.0), jnp.where(sub == d, s_b, 0.0))

    isub = jax.lax.broadcasted_iota(jnp.int32, (n, n), 0)
    jlan = jax.lax.broadcasted_iota(jnp.int32, (n, n), 1)
    sr = jnp.broadcast_to(s_row, (n, n))
    sc = jnp.broadcast_to(s_col, (n, n))
    cmp = (sr > sc) | ((sr == sc) & (jlan < isub))
    rank = jnp.sum(cmp.astype(jnp.float32), axis=1, keepdims=True)

    kf = jax.lax.broadcasted_iota(jnp.int32, (n, n), 1).astype(jnp.float32)
    PT = (jnp.broadcast_to(rank, (n, n)) == kf).astype(jnp.float32)
    O = jnp.dot(G, PT, preferred_element_type=jnp.float32)

    kmask = jax.lax.broadcasted_iota(jnp.int32, (m, n), 1) < k
    Om = jnp.where(kmask, O, 0.0)
    sq = jnp.sum(Om * Om, axis=1, keepdims=True)
    den = jnp.maximum(jnp.sqrt(sq), 1e-12)
    desc = O[0:d, :] / den[0:d, :]
    od_ref[0] = desc[:, 0:k]
    os_ref[0] = O[d:d + 1, 0:k]


def kernel(features, att_w, att_b, proj_w, proj_b, bn_gamma, bn_beta,
           bn_mean, bn_var, num_keypoints):
    B, C, H, W = features.shape
    D = proj_w.shape[0]
    N = H * W
    K = min(1000, N)
    eps = 1e-5
    M = D + 8

    x = features.reshape(B, C, N)
    att = jax.nn.sigmoid(jnp.einsum('bchw,oc->bohw', features, att_w)
                         + att_b[None, :, None, None])
    s3 = att.reshape(B, 1, N)
    scale = bn_gamma / jnp.sqrt(bn_var + eps)
    w_loc = proj_w * scale[:, None]
    b_loc = (proj_b - bn_mean) * scale + bn_beta
    w_all = jnp.concatenate(
        [w_loc, jnp.zeros((M - D, C), jnp.float32)], axis=0)
    b_all = jnp.concatenate(
        [b_loc, jnp.zeros((M - D,), jnp.float32)], axis=0)
    b_all = jnp.broadcast_to(b_all[:, None], (M, N))

    body = functools.partial(_body, d=D, n=N, k=K)
    out = pl.pallas_call(
        body,
        grid=(B,),
        in_specs=[
            pl.BlockSpec((1, C, N), lambda b: (b, 0, 0)),
            pl.BlockSpec((M, C), lambda b: (0, 0)),
            pl.BlockSpec((M, N), lambda b: (0, 0)),
            pl.BlockSpec((1, 1, N), lambda b: (b, 0, 0)),
        ],
        out_specs=[
            pl.BlockSpec((1, D, K), lambda b: (b, 0, 0)),
            pl.BlockSpec((1, 1, K), lambda b: (b, 0, 0)),
        ],
        out_shape=[
            jax.ShapeDtypeStruct((B, D, K), jnp.float32),
            jax.ShapeDtypeStruct((B, 1, K), jnp.float32),
        ],
        compiler_params=pltpu.CompilerParams(
            dimension_semantics=("parallel",)),
    )(x, w_all, b_all, s3)

    local_desc, scores3 = out
    return (local_desc, scores3[:, 0, :])
